# Initial kernel scaffold; baseline (speedup 1.0000x reference)
import jax
import jax.numpy as jnp
from jax.experimental import pallas as pl


def kernel(z, embedding):
    raise NotImplementedError("write your pallas kernel here")


# speedup vs baseline: 1.0170x; 1.0170x over previous
import functools

import jax
import jax.numpy as jnp
from jax import lax
from jax.experimental import pallas as pl
from jax.experimental.pallas import tpu as pltpu
from jax.experimental.pallas import tpu_sc as plsc

_N_ROWS = 8192
_N_CODES = 8192
_DIM = 32
_BM = 1024
_BK = 2048
_BETA = 0.25


def _bf16_round(x):
    return x.astype(jnp.bfloat16).astype(jnp.float32)


def _vq_dist_argmin_body(z_ref, e_ref, zn_ref, en_ref, idx_ref, loss_ref,
                         acc_ref, pick_ref, vsel_ref, lsum_ref):
    i = pl.program_id(0)
    j = pl.program_id(1)
    nj = pl.num_programs(1)

    zb = z_ref[...]
    eb = e_ref[...]
    zn = zn_ref[...]
    en = en_ref[...]

    t = lax.dot_general(eb, zb, (((1,), (1,)), ((), ())),
                        preferred_element_type=jnp.float32)
    d = (zn + en) - 2.0 * t

    cur_min = jnp.min(d, axis=0, keepdims=True)
    ids = lax.broadcasted_iota(jnp.int32, (_BK, _BM), 0) + j * _BK
    cur_idx = jnp.min(jnp.where(d == cur_min, ids, jnp.int32(2**31 - 1)),
                      axis=0, keepdims=True)

    @pl.when(j == 0)
    def _():
        acc_ref[...] = _bf16_round(cur_min)
        pick_ref[...] = cur_idx
        vsel_ref[...] = cur_min

    @pl.when(j > 0)
    def _():
        beat = cur_min < acc_ref[...]
        acc_ref[...] = jnp.where(beat, _bf16_round(cur_min), acc_ref[...])
        pick_ref[...] = jnp.where(beat, cur_idx, pick_ref[...])
        vsel_ref[...] = jnp.where(beat, cur_min, vsel_ref[...])

    @pl.when(j == nj - 1)
    def _():
        idx_ref[...] = pick_ref[...].reshape(1, 1, _BM)
        blk = jnp.sum(vsel_ref[...])

        @pl.when(i == 0)
        def _():
            lsum_ref[0, 0] = blk

        @pl.when(i > 0)
        def _():
            lsum_ref[0, 0] = lsum_ref[0, 0] + blk

        @pl.when(i == pl.num_programs(0) - 1)
        def _():
            m = lsum_ref[0, 0] / jnp.float32(_N_ROWS * _DIM)
            loss_ref[0, 0] = jnp.float32(_BETA) * m + m


def _dist_argmin(zb, eb, zn, en):
    grid = (_N_ROWS // _BM, _N_CODES // _BK)
    idx3, loss = pl.pallas_call(
        _vq_dist_argmin_body,
        grid=grid,
        in_specs=[
            pl.BlockSpec((_BM, _DIM), lambda i, j: (i, 0)),
            pl.BlockSpec((_BK, _DIM), lambda i, j: (j, 0)),
            pl.BlockSpec((1, _BM), lambda i, j: (0, i)),
            pl.BlockSpec((_BK, 1), lambda i, j: (j, 0)),
        ],
        out_specs=[
            pl.BlockSpec((1, 1, _BM), lambda i, j: (i, 0, 0)),
            pl.BlockSpec(memory_space=pltpu.SMEM),
        ],
        out_shape=[
            jax.ShapeDtypeStruct((_N_ROWS // _BM, 1, _BM), jnp.int32),
            jax.ShapeDtypeStruct((1, 1), jnp.float32),
        ],
        scratch_shapes=[
            pltpu.VMEM((1, _BM), jnp.float32),
            pltpu.VMEM((1, _BM), jnp.int32),
            pltpu.VMEM((1, _BM), jnp.float32),
            pltpu.SMEM((1, 1), jnp.float32),
        ],
        compiler_params=pltpu.CompilerParams(
            dimension_semantics=("arbitrary", "arbitrary")),
    )(zb, eb, zn, en)
    return idx3.reshape(_N_ROWS), loss


_DPAD = 128


@functools.lru_cache(maxsize=1)
def _make_sc_gather():
    info = plsc.get_sparse_core_info()
    nc, ns = info.num_cores, info.num_subcores
    nw = nc * ns
    b_per_w = _N_ROWS // nw
    mesh = plsc.VectorSubcoreMesh(core_axis_name="c", subcore_axis_name="s")

    @functools.partial(
        pl.kernel, mesh=mesh,
        out_type=jax.ShapeDtypeStruct((_N_ROWS, _DPAD), jnp.float32),
        scratch_types=[
            pltpu.VMEM((b_per_w,), jnp.int32),
            pltpu.VMEM((b_per_w, _DPAD), jnp.float32),
            pltpu.SemaphoreType.DMA,
        ],
    )
    def gather(table_hbm, idx_hbm, out_hbm, idx_v, rows_v, sem):
        wid = lax.axis_index("s") * nc + lax.axis_index("c")
        base = wid * b_per_w
        pltpu.sync_copy(idx_hbm.at[pl.ds(base, b_per_w)], idx_v)
        pltpu.async_copy(table_hbm.at[idx_v], rows_v, sem).wait()
        pltpu.sync_copy(rows_v, out_hbm.at[pl.ds(base, b_per_w)])

    return gather


def kernel(z, embedding):
    b, c, h, w = z.shape
    z_flat = jnp.transpose(z, (0, 2, 3, 1)).reshape(-1, c)
    zn = jnp.sum(z_flat ** 2, axis=1, keepdims=True)
    en = jnp.sum(embedding ** 2, axis=1)
    zb = z_flat.astype(jnp.bfloat16)
    eb = embedding.astype(jnp.bfloat16)

    idx, loss = _dist_argmin(zb, eb,
                             zn.reshape(1, _N_ROWS), en.reshape(_N_CODES, 1))

    emb_pad = jnp.pad(embedding, ((0, 0), (0, _DPAD - _DIM)))
    zq_rows = _make_sc_gather()(emb_pad, idx)[:, :_DIM]

    zq = z_flat + lax.stop_gradient(zq_rows - z_flat)
    zq = jnp.transpose(zq.reshape(b, h, w, c), (0, 3, 1, 2))
    return (zq, loss.reshape(()))


# speedup vs baseline: 1.1520x; 1.1327x over previous
import functools

import jax
import jax.numpy as jnp
from jax import lax
from jax.experimental import pallas as pl
from jax.experimental.pallas import tpu as pltpu
from jax.experimental.pallas import tpu_sc as plsc

_N_ROWS = 8192
_N_CODES = 8192
_DIM = 32
_BM = 4096
_BK = 1024
_BETA = 0.25


def _bf16_round(x):
    return x.astype(jnp.bfloat16).astype(jnp.float32)


def _vq_dist_argmin_body(z_ref, e_ref, zn_ref, en_ref, idx_ref, loss_ref,
                         acc_ref, pick_ref, vsel_ref, run_v_ref, run_i_ref,
                         lsum_ref):
    i = pl.program_id(0)
    j = pl.program_id(1)
    nj = pl.num_programs(1)

    zb = z_ref[...]
    eb = e_ref[...]
    zn = zn_ref[...]
    en = en_ref[...]

    t = lax.dot_general(eb, zb, (((1,), (1,)), ((), ())),
                        preferred_element_type=jnp.float32)
    d = (zn + en) + t

    cur_min = jnp.min(d, axis=0, keepdims=True)
    ids = lax.broadcasted_iota(jnp.int32, (_BK, _BM), 0)
    cur_idx = jnp.min(jnp.where(d == cur_min, ids, jnp.int32(2**30)),
                      axis=0, keepdims=True) + j * _BK

    @pl.when(j % 2 == 0)
    def _():
        run_v_ref[...] = cur_min
        run_i_ref[...] = cur_idx

    @pl.when(j % 2 == 1)
    def _():
        better = cur_min < run_v_ref[...]
        run_v = jnp.where(better, cur_min, run_v_ref[...])
        run_i = jnp.where(better, cur_idx, run_i_ref[...])

        @pl.when(j == 1)
        def _():
            acc_ref[...] = _bf16_round(run_v)
            pick_ref[...] = run_i
            vsel_ref[...] = run_v

        @pl.when(j > 1)
        def _():
            beat = run_v < acc_ref[...]
            acc_ref[...] = jnp.where(beat, _bf16_round(run_v), acc_ref[...])
            pick_ref[...] = jnp.where(beat, run_i, pick_ref[...])
            vsel_ref[...] = jnp.where(beat, run_v, vsel_ref[...])

    @pl.when(j == nj - 1)
    def _():
        idx_ref[...] = pick_ref[...].reshape(1, 1, _BM)
        blk = jnp.sum(vsel_ref[...])

        @pl.when(i == 0)
        def _():
            lsum_ref[0, 0] = blk

        @pl.when(i > 0)
        def _():
            lsum_ref[0, 0] = lsum_ref[0, 0] + blk

        @pl.when(i == pl.num_programs(0) - 1)
        def _():
            m = lsum_ref[0, 0] / jnp.float32(_N_ROWS * _DIM)
            loss_ref[0, 0] = jnp.float32(_BETA) * m + m


def _dist_argmin(zb, eb, zn, en):
    grid = (_N_ROWS // _BM, _N_CODES // _BK)
    idx3, loss = pl.pallas_call(
        _vq_dist_argmin_body,
        grid=grid,
        in_specs=[
            pl.BlockSpec((_BM, _DIM), lambda i, j: (i, 0)),
            pl.BlockSpec((_BK, _DIM), lambda i, j: (j, 0)),
            pl.BlockSpec((1, _BM), lambda i, j: (0, i)),
            pl.BlockSpec((_BK, 1), lambda i, j: (j, 0)),
        ],
        out_specs=[
            pl.BlockSpec((1, 1, _BM), lambda i, j: (i, 0, 0)),
            pl.BlockSpec(memory_space=pltpu.SMEM),
        ],
        out_shape=[
            jax.ShapeDtypeStruct((_N_ROWS // _BM, 1, _BM), jnp.int32),
            jax.ShapeDtypeStruct((1, 1), jnp.float32),
        ],
        scratch_shapes=[
            pltpu.VMEM((1, _BM), jnp.float32),
            pltpu.VMEM((1, _BM), jnp.int32),
            pltpu.VMEM((1, _BM), jnp.float32),
            pltpu.VMEM((1, _BM), jnp.float32),
            pltpu.VMEM((1, _BM), jnp.int32),
            pltpu.SMEM((1, 1), jnp.float32),
        ],
        compiler_params=pltpu.CompilerParams(
            dimension_semantics=("arbitrary", "arbitrary")),
    )(zb, eb, zn, en)
    return idx3.reshape(_N_ROWS), loss


_DPAD = 128


@functools.lru_cache(maxsize=1)
def _make_sc_gather():
    info = plsc.get_sparse_core_info()
    nc, ns = info.num_cores, info.num_subcores
    nw = nc * ns
    b_per_w = _N_ROWS // nw
    mesh = plsc.VectorSubcoreMesh(core_axis_name="c", subcore_axis_name="s")

    @functools.partial(
        pl.kernel, mesh=mesh,
        out_type=jax.ShapeDtypeStruct((_N_ROWS, _DPAD), jnp.float32),
        scratch_types=[
            pltpu.VMEM((b_per_w,), jnp.int32),
            pltpu.VMEM((b_per_w, _DPAD), jnp.float32),
            pltpu.SemaphoreType.DMA,
        ],
    )
    def gather(table_hbm, idx_hbm, out_hbm, idx_v, rows_v, sem):
        wid = lax.axis_index("s") * nc + lax.axis_index("c")
        base = wid * b_per_w
        pltpu.sync_copy(idx_hbm.at[pl.ds(base, b_per_w)], idx_v)
        pltpu.async_copy(table_hbm.at[idx_v], rows_v, sem).wait()
        pltpu.sync_copy(rows_v, out_hbm.at[pl.ds(base, b_per_w)])

    return gather


def kernel(z, embedding):
    b, c, h, w = z.shape
    z_flat = jnp.transpose(z, (0, 2, 3, 1)).reshape(-1, c)
    zn = jnp.sum(z_flat ** 2, axis=1, keepdims=True)
    en = jnp.sum(embedding ** 2, axis=1)
    zb = z_flat.astype(jnp.bfloat16)
    eb = embedding.astype(jnp.bfloat16) * jnp.bfloat16(-2.0)

    idx, loss = _dist_argmin(zb, eb,
                             zn.reshape(1, _N_ROWS), en.reshape(_N_CODES, 1))

    emb_pad = jnp.pad(embedding, ((0, 0), (0, _DPAD - _DIM)))
    zq_rows = _make_sc_gather()(emb_pad, idx)[:, :_DIM]

    zq = z_flat + lax.stop_gradient(zq_rows - z_flat)
    zq = jnp.transpose(zq.reshape(b, h, w, c), (0, 3, 1, 2))
    return (zq, loss.reshape(()))


# speedup vs baseline: 1.1530x; 1.0009x over previous
import functools

import jax
import jax.numpy as jnp
from jax import lax
from jax.experimental import pallas as pl
from jax.experimental.pallas import tpu as pltpu
from jax.experimental.pallas import tpu_sc as plsc

_N_ROWS = 8192
_N_CODES = 8192
_DIM = 32
_BM = 4096
_BK = 1024
_CHUNK = 2048
_SUBS = _CHUNK // _BK
_BETA = 0.25


def _bf16_round(x):
    return x.astype(jnp.bfloat16).astype(jnp.float32)


def _vq_dist_argmin_body(z_ref, e_ref, zn_ref, en_ref, idx_ref, loss_ref,
                         acc_ref, pick_ref, vsel_ref, run_v_ref, run_i_ref,
                         lsum_ref):
    i = pl.program_id(0)
    j = pl.program_id(1)
    nj = pl.num_programs(1)

    zb = z_ref[...]
    eb = e_ref[...]
    zn = zn_ref[...]
    en = en_ref[...]

    t = lax.dot_general(eb, zb, (((1,), (1,)), ((), ())),
                        preferred_element_type=jnp.float32)
    d = (zn + en) + t

    cur_min = jnp.min(d, axis=0, keepdims=True)
    ids = lax.broadcasted_iota(jnp.int32, (_BK, _BM), 0)
    cur_idx = jnp.min(jnp.where(d == cur_min, ids, jnp.int32(2**30)),
                      axis=0, keepdims=True) + j * _BK

    @pl.when(j % _SUBS == 0)
    def _():
        run_v_ref[...] = cur_min
        run_i_ref[...] = cur_idx

    @pl.when(j % _SUBS > 0)
    def _():
        better = cur_min < run_v_ref[...]
        run_v_ref[...] = jnp.where(better, cur_min, run_v_ref[...])
        run_i_ref[...] = jnp.where(better, cur_idx, run_i_ref[...])

    @pl.when(j % _SUBS == _SUBS - 1)
    def _():
        run_v = run_v_ref[...]
        run_i = run_i_ref[...]

        @pl.when(j == _SUBS - 1)
        def _():
            acc_ref[...] = _bf16_round(run_v)
            pick_ref[...] = run_i
            vsel_ref[...] = run_v

        @pl.when(j > _SUBS - 1)
        def _():
            beat = run_v < acc_ref[...]
            acc_ref[...] = jnp.where(beat, _bf16_round(run_v), acc_ref[...])
            pick_ref[...] = jnp.where(beat, run_i, pick_ref[...])
            vsel_ref[...] = jnp.where(beat, run_v, vsel_ref[...])

    @pl.when(j == nj - 1)
    def _():
        idx_ref[...] = pick_ref[...].reshape(1, 1, _BM)
        blk = jnp.sum(vsel_ref[...])

        @pl.when(i == 0)
        def _():
            lsum_ref[0, 0] = blk

        @pl.when(i > 0)
        def _():
            lsum_ref[0, 0] = lsum_ref[0, 0] + blk

        @pl.when(i == pl.num_programs(0) - 1)
        def _():
            m = lsum_ref[0, 0] / jnp.float32(_N_ROWS * _DIM)
            loss_ref[0, 0] = jnp.float32(_BETA) * m + m


def _dist_argmin(zb, eb, zn, en):
    grid = (_N_ROWS // _BM, _N_CODES // _BK)
    idx3, loss = pl.pallas_call(
        _vq_dist_argmin_body,
        grid=grid,
        in_specs=[
            pl.BlockSpec((_BM, _DIM), lambda i, j: (i, 0)),
            pl.BlockSpec((_BK, _DIM), lambda i, j: (j, 0)),
            pl.BlockSpec((1, _BM), lambda i, j: (0, i)),
            pl.BlockSpec((_BK, 1), lambda i, j: (j, 0)),
        ],
        out_specs=[
            pl.BlockSpec((1, 1, _BM), lambda i, j: (i, 0, 0)),
            pl.BlockSpec(memory_space=pltpu.SMEM),
        ],
        out_shape=[
            jax.ShapeDtypeStruct((_N_ROWS // _BM, 1, _BM), jnp.int32),
            jax.ShapeDtypeStruct((1, 1), jnp.float32),
        ],
        scratch_shapes=[
            pltpu.VMEM((1, _BM), jnp.float32),
            pltpu.VMEM((1, _BM), jnp.int32),
            pltpu.VMEM((1, _BM), jnp.float32),
            pltpu.VMEM((1, _BM), jnp.float32),
            pltpu.VMEM((1, _BM), jnp.int32),
            pltpu.SMEM((1, 1), jnp.float32),
        ],
        compiler_params=pltpu.CompilerParams(
            dimension_semantics=("arbitrary", "arbitrary")),
    )(zb, eb, zn, en)
    return idx3.reshape(_N_ROWS), loss


_DPAD = 128


@functools.lru_cache(maxsize=1)
def _make_sc_gather():
    info = plsc.get_sparse_core_info()
    nc, ns = info.num_cores, info.num_subcores
    nw = nc * ns
    b_per_w = _N_ROWS // nw
    mesh = plsc.VectorSubcoreMesh(core_axis_name="c", subcore_axis_name="s")

    @functools.partial(
        pl.kernel, mesh=mesh,
        out_type=jax.ShapeDtypeStruct((_N_ROWS, _DPAD), jnp.float32),
        scratch_types=[
            pltpu.VMEM((b_per_w,), jnp.int32),
            pltpu.VMEM((b_per_w, _DPAD), jnp.float32),
            pltpu.SemaphoreType.DMA,
        ],
    )
    def gather(table_hbm, idx_hbm, out_hbm, idx_v, rows_v, sem):
        wid = lax.axis_index("s") * nc + lax.axis_index("c")
        base = wid * b_per_w
        pltpu.sync_copy(idx_hbm.at[pl.ds(base, b_per_w)], idx_v)
        pltpu.async_copy(table_hbm.at[idx_v], rows_v, sem).wait()
        pltpu.sync_copy(rows_v, out_hbm.at[pl.ds(base, b_per_w)])

    return gather


def kernel(z, embedding):
    b, c, h, w = z.shape
    z_flat = jnp.transpose(z, (0, 2, 3, 1)).reshape(-1, c)
    zn = jnp.sum(z_flat ** 2, axis=1, keepdims=True)
    en = jnp.sum(embedding ** 2, axis=1)
    zb = z_flat.astype(jnp.bfloat16)
    eb = embedding.astype(jnp.bfloat16) * jnp.bfloat16(-2.0)

    idx, loss = _dist_argmin(zb, eb,
                             zn.reshape(1, _N_ROWS), en.reshape(_N_CODES, 1))

    emb_pad = jnp.pad(embedding, ((0, 0), (0, _DPAD - _DIM)))
    zq_rows = _make_sc_gather()(emb_pad, idx)[:, :_DIM]

    zq = z_flat + lax.stop_gradient(zq_rows - z_flat)
    zq = jnp.transpose(zq.reshape(b, h, w, c), (0, 3, 1, 2))
    return (zq, loss.reshape(()))


# speedup vs baseline: 1.2061x; 1.0461x over previous
import functools

import jax
import jax.numpy as jnp
from jax import lax
from jax.experimental import pallas as pl
from jax.experimental.pallas import tpu as pltpu
from jax.experimental.pallas import tpu_sc as plsc

_N_ROWS = 8192
_N_CODES = 8192
_DIM = 32
_BM = 8192
_BK = 2048
_CHUNK = 2048
_SUBS = _CHUNK // _BK
_BETA = 0.25


def _bf16_round(x):
    return x.astype(jnp.bfloat16).astype(jnp.float32)


def _vq_dist_argmin_body(z_ref, e_ref, zn_ref, en_ref, idx_ref, loss_ref,
                         acc_ref, pick_ref, vsel_ref, run_v_ref, run_i_ref,
                         lsum_ref):
    i = pl.program_id(0)
    j = pl.program_id(1)
    nj = pl.num_programs(1)

    zb = z_ref[...]
    eb = e_ref[...]
    zn = zn_ref[...]
    en = en_ref[...]

    t = lax.dot_general(eb, zb, (((1,), (1,)), ((), ())),
                        preferred_element_type=jnp.float32)
    d = (zn + en) + t

    cur_min = jnp.min(d, axis=0, keepdims=True)
    ids = lax.broadcasted_iota(jnp.int32, (_BK, _BM), 0)
    cur_idx = jnp.min(jnp.where(d == cur_min, ids, jnp.int32(2**30)),
                      axis=0, keepdims=True) + j * _BK

    @pl.when(j % _SUBS == 0)
    def _():
        run_v_ref[...] = cur_min
        run_i_ref[...] = cur_idx

    @pl.when(j % _SUBS > 0)
    def _():
        better = cur_min < run_v_ref[...]
        run_v_ref[...] = jnp.where(better, cur_min, run_v_ref[...])
        run_i_ref[...] = jnp.where(better, cur_idx, run_i_ref[...])

    @pl.when(j % _SUBS == _SUBS - 1)
    def _():
        run_v = run_v_ref[...]
        run_i = run_i_ref[...]

        @pl.when(j == _SUBS - 1)
        def _():
            acc_ref[...] = _bf16_round(run_v)
            pick_ref[...] = run_i
            vsel_ref[...] = run_v

        @pl.when(j > _SUBS - 1)
        def _():
            beat = run_v < acc_ref[...]
            acc_ref[...] = jnp.where(beat, _bf16_round(run_v), acc_ref[...])
            pick_ref[...] = jnp.where(beat, run_i, pick_ref[...])
            vsel_ref[...] = jnp.where(beat, run_v, vsel_ref[...])

    @pl.when(j == nj - 1)
    def _():
        idx_ref[...] = pick_ref[...].reshape(1, 1, _BM)
        blk = jnp.sum(vsel_ref[...])

        @pl.when(i == 0)
        def _():
            lsum_ref[0, 0] = blk

        @pl.when(i > 0)
        def _():
            lsum_ref[0, 0] = lsum_ref[0, 0] + blk

        @pl.when(i == pl.num_programs(0) - 1)
        def _():
            m = lsum_ref[0, 0] / jnp.float32(_N_ROWS * _DIM)
            loss_ref[0, 0] = jnp.float32(_BETA) * m + m


def _dist_argmin(zb, eb, zn, en):
    grid = (_N_ROWS // _BM, _N_CODES // _BK)
    idx3, loss = pl.pallas_call(
        _vq_dist_argmin_body,
        grid=grid,
        in_specs=[
            pl.BlockSpec((_BM, _DIM), lambda i, j: (i, 0)),
            pl.BlockSpec((_BK, _DIM), lambda i, j: (j, 0)),
            pl.BlockSpec((1, _BM), lambda i, j: (0, i)),
            pl.BlockSpec((_BK, 1), lambda i, j: (j, 0)),
        ],
        out_specs=[
            pl.BlockSpec((1, 1, _BM), lambda i, j: (i, 0, 0)),
            pl.BlockSpec(memory_space=pltpu.SMEM),
        ],
        out_shape=[
            jax.ShapeDtypeStruct((_N_ROWS // _BM, 1, _BM), jnp.int32),
            jax.ShapeDtypeStruct((1, 1), jnp.float32),
        ],
        scratch_shapes=[
            pltpu.VMEM((1, _BM), jnp.float32),
            pltpu.VMEM((1, _BM), jnp.int32),
            pltpu.VMEM((1, _BM), jnp.float32),
            pltpu.VMEM((1, _BM), jnp.float32),
            pltpu.VMEM((1, _BM), jnp.int32),
            pltpu.SMEM((1, 1), jnp.float32),
        ],
        compiler_params=pltpu.CompilerParams(
            dimension_semantics=("arbitrary", "arbitrary")),
    )(zb, eb, zn, en)
    return idx3.reshape(_N_ROWS), loss


_DPAD = 128


@functools.lru_cache(maxsize=1)
def _make_sc_gather():
    info = plsc.get_sparse_core_info()
    nc, ns = info.num_cores, info.num_subcores
    nw = nc * ns
    b_per_w = _N_ROWS // nw
    mesh = plsc.VectorSubcoreMesh(core_axis_name="c", subcore_axis_name="s")

    @functools.partial(
        pl.kernel, mesh=mesh,
        out_type=jax.ShapeDtypeStruct((_N_ROWS, _DPAD), jnp.float32),
        scratch_types=[
            pltpu.VMEM((b_per_w,), jnp.int32),
            pltpu.VMEM((b_per_w, _DPAD), jnp.float32),
            pltpu.SemaphoreType.DMA,
        ],
    )
    def gather(table_hbm, idx_hbm, out_hbm, idx_v, rows_v, sem):
        wid = lax.axis_index("s") * nc + lax.axis_index("c")
        base = wid * b_per_w
        pltpu.sync_copy(idx_hbm.at[pl.ds(base, b_per_w)], idx_v)
        pltpu.async_copy(table_hbm.at[idx_v], rows_v, sem).wait()
        pltpu.sync_copy(rows_v, out_hbm.at[pl.ds(base, b_per_w)])

    return gather


def kernel(z, embedding):
    b, c, h, w = z.shape
    z_flat = jnp.transpose(z, (0, 2, 3, 1)).reshape(-1, c)
    zn = jnp.sum(z_flat ** 2, axis=1, keepdims=True)
    en = jnp.sum(embedding ** 2, axis=1)
    zb = z_flat.astype(jnp.bfloat16)
    eb = embedding.astype(jnp.bfloat16) * jnp.bfloat16(-2.0)

    idx, loss = _dist_argmin(zb, eb,
                             zn.reshape(1, _N_ROWS), en.reshape(_N_CODES, 1))

    emb_pad = jnp.pad(embedding, ((0, 0), (0, _DPAD - _DIM)))
    zq_rows = _make_sc_gather()(emb_pad, idx)[:, :_DIM]

    zq = z_flat + lax.stop_gradient(zq_rows - z_flat)
    zq = jnp.transpose(zq.reshape(b, h, w, c), (0, 3, 1, 2))
    return (zq, loss.reshape(()))
